# baseline (speedup 1.0000x reference)
import functools

import jax
import jax.numpy as jnp
from jax import lax
from jax.experimental import pallas as pl
from jax.experimental.pallas import tpu as pltpu
from jax.experimental.pallas import tpu_sc as plsc

BATCH = 16384
N_IN = 16
N_RULES = 64
NUM_CORES = 2
NUM_SUBCORES = 16
NUM_WORKERS = NUM_CORES * NUM_SUBCORES
COLS_PER_WORKER = BATCH // NUM_WORKERS
LANES = 16
GROUPS = COLS_PER_WORKER // LANES


HALF = COLS_PER_WORKER // 2
QUARTER = COLS_PER_WORKER // 4
Q_GROUPS = QUARTER // LANES


def _body(xt_hbm, outt_hbm, in_v, out_v,
          isem0, isem1, osem0, osem1, osem2, osem3):
    wid = lax.axis_index("s") * NUM_CORES + lax.axis_index("c")
    base = wid * COLS_PER_WORKER

    hin0 = pltpu.async_copy(xt_hbm.at[:, pl.ds(base, HALF)],
                            in_v.at[:, pl.ds(0, HALF)], isem0)
    hin1 = pltpu.async_copy(xt_hbm.at[:, pl.ds(base + HALF, HALF)],
                            in_v.at[:, pl.ds(HALF, HALF)], isem1)
    hin0.wait()

    def group(g, carry):
        col = g * LANES
        vals = [in_v[p, pl.ds(col, LANES)] for p in range(N_IN)]
        a, c = vals[:8], vals[8:]
        for i in range(8):
            for j in range(8):
                out_v[i * 8 + j, pl.ds(col, LANES)] = jnp.minimum(a[i], c[j])
        return carry

    osems = (osem0, osem1, osem2, osem3)
    pending = []
    for q in range(4):
        if q == 2:
            hin1.wait()
        lax.fori_loop(q * Q_GROUPS, (q + 1) * Q_GROUPS, group, 0)
        pending.append(pltpu.async_copy(
            out_v.at[:, pl.ds(q * QUARTER, QUARTER)],
            outt_hbm.at[:, pl.ds(base + q * QUARTER, QUARTER)], osems[q]))
    for h in pending:
        h.wait()


@functools.partial(jax.jit, static_argnames=())
def _run(xt):
    mesh = plsc.VectorSubcoreMesh(
        core_axis_name="c", subcore_axis_name="s",
        num_cores=NUM_CORES, num_subcores=NUM_SUBCORES,
    )
    k = pl.kernel(
        _body,
        out_type=jax.ShapeDtypeStruct((N_RULES, BATCH), jnp.float32),
        mesh=mesh,
        scratch_types=[
            pltpu.VMEM((N_IN, COLS_PER_WORKER), jnp.float32),
            pltpu.VMEM((N_RULES, COLS_PER_WORKER), jnp.float32),
            pltpu.SemaphoreType.DMA,
            pltpu.SemaphoreType.DMA,
            pltpu.SemaphoreType.DMA,
            pltpu.SemaphoreType.DMA,
            pltpu.SemaphoreType.DMA,
            pltpu.SemaphoreType.DMA,
        ],
        compiler_params=pltpu.CompilerParams(needs_layout_passes=False),
    )
    return k(xt)


def kernel(x):
    xt = x.transpose(1, 2, 0).reshape(N_IN, BATCH)
    return _run(xt).T


# speedup vs baseline: 1.0161x; 1.0161x over previous
import functools

import jax
import jax.numpy as jnp
from jax import lax
from jax.experimental import pallas as pl
from jax.experimental.pallas import tpu as pltpu
from jax.experimental.pallas import tpu_sc as plsc

BATCH = 16384
N_IN = 16
N_RULES = 64
NUM_CORES = 2
NUM_SUBCORES = 16
NUM_WORKERS = NUM_CORES * NUM_SUBCORES
COLS_PER_WORKER = BATCH // NUM_WORKERS
LANES = 16
GROUPS = COLS_PER_WORKER // LANES


HALF = COLS_PER_WORKER // 2
QUARTER = COLS_PER_WORKER // 4
Q_GROUPS = QUARTER // LANES


def _body(xt_hbm, outt_hbm, in_v, out_v, isem0, isem1, osem0, osem1):
    wid = lax.axis_index("s") * NUM_CORES + lax.axis_index("c")
    base = wid * COLS_PER_WORKER

    hin0 = pltpu.async_copy(xt_hbm.at[:, pl.ds(base, HALF)],
                            in_v.at[:, pl.ds(0, HALF)], isem0)
    hin1 = pltpu.async_copy(xt_hbm.at[:, pl.ds(base + HALF, HALF)],
                            in_v.at[:, pl.ds(HALF, HALF)], isem1)
    hin0.wait()

    def group(g, carry):
        col = g * LANES
        vals = [in_v[p, pl.ds(col, LANES)] for p in range(N_IN)]
        a, c = vals[:8], vals[8:]
        for i in range(8):
            for j in range(8):
                out_v[i * 8 + j, pl.ds(col, LANES)] = jnp.minimum(a[i], c[j])
        return carry

    lax.fori_loop(0, 2 * Q_GROUPS, group, 0)
    h0 = pltpu.async_copy(out_v.at[:, pl.ds(0, HALF)],
                          outt_hbm.at[:, pl.ds(base, HALF)], osem0)
    hin1.wait()
    lax.fori_loop(2 * Q_GROUPS, 4 * Q_GROUPS, group, 0)
    h1 = pltpu.async_copy(out_v.at[:, pl.ds(HALF, HALF)],
                          outt_hbm.at[:, pl.ds(base + HALF, HALF)], osem1)
    h0.wait()
    h1.wait()


@functools.partial(jax.jit, static_argnames=())
def _run(xt):
    mesh = plsc.VectorSubcoreMesh(
        core_axis_name="c", subcore_axis_name="s",
        num_cores=NUM_CORES, num_subcores=NUM_SUBCORES,
    )
    k = pl.kernel(
        _body,
        out_type=jax.ShapeDtypeStruct((N_RULES, BATCH), jnp.float32),
        mesh=mesh,
        scratch_types=[
            pltpu.VMEM((N_IN, COLS_PER_WORKER), jnp.float32),
            pltpu.VMEM((N_RULES, COLS_PER_WORKER), jnp.float32),
            pltpu.SemaphoreType.DMA,
            pltpu.SemaphoreType.DMA,
            pltpu.SemaphoreType.DMA,
            pltpu.SemaphoreType.DMA,
        ],
        compiler_params=pltpu.CompilerParams(needs_layout_passes=False),
    )
    return k(xt)


def kernel(x):
    xt = x.transpose(1, 2, 0).reshape(N_IN, BATCH)
    return _run(xt).T


# speedup vs baseline: 1.0168x; 1.0007x over previous
import functools

import jax
import jax.numpy as jnp
from jax import lax
from jax.experimental import pallas as pl
from jax.experimental.pallas import tpu as pltpu
from jax.experimental.pallas import tpu_sc as plsc

BATCH = 16384
N_IN = 16
N_RULES = 64
NUM_CORES = 2
NUM_SUBCORES = 16
NUM_WORKERS = NUM_CORES * NUM_SUBCORES
COLS_PER_WORKER = BATCH // NUM_WORKERS
LANES = 16
GROUPS = COLS_PER_WORKER // LANES


HALF = COLS_PER_WORKER // 2
QUARTER = COLS_PER_WORKER // 4
Q_GROUPS = QUARTER // LANES


def _body(xt_hbm, outt_hbm, in_v, out_v, isem0, isem1, osem0, osem1):
    wid = lax.axis_index("s") * NUM_CORES + lax.axis_index("c")
    base = wid * COLS_PER_WORKER

    hin0 = pltpu.async_copy(xt_hbm.at[:, pl.ds(base, HALF)],
                            in_v.at[:, pl.ds(0, HALF)], isem0)
    hin1 = pltpu.async_copy(xt_hbm.at[:, pl.ds(base + HALF, HALF)],
                            in_v.at[:, pl.ds(HALF, HALF)], isem1)
    hin0.wait()

    def group(g, carry):
        col = g * LANES
        vals = [in_v[p, pl.ds(col, LANES)] for p in range(N_IN)]
        a, c = vals[:8], vals[8:]
        for i in range(8):
            for j in range(8):
                out_v[i * 8 + j, pl.ds(col, LANES)] = jnp.minimum(a[i], c[j])
        return carry

    lax.fori_loop(0, 2 * Q_GROUPS, group, 0)
    h0 = pltpu.async_copy(out_v.at[:, pl.ds(0, HALF)],
                          outt_hbm.at[:, pl.ds(base, HALF)], osem0)
    hin1.wait()
    lax.fori_loop(2 * Q_GROUPS, 4 * Q_GROUPS, group, 0)
    h1 = pltpu.async_copy(out_v.at[:, pl.ds(HALF, HALF)],
                          outt_hbm.at[:, pl.ds(base + HALF, HALF)], osem1)
    h0.wait()
    h1.wait()


@functools.partial(jax.jit, static_argnames=())
def _run(xt):
    mesh = plsc.VectorSubcoreMesh(
        core_axis_name="c", subcore_axis_name="s",
        num_cores=NUM_CORES, num_subcores=NUM_SUBCORES,
    )
    k = pl.kernel(
        _body,
        out_type=jax.ShapeDtypeStruct((N_RULES, BATCH), jnp.float32),
        mesh=mesh,
        scratch_types=[
            pltpu.VMEM((N_IN, COLS_PER_WORKER), jnp.float32),
            pltpu.VMEM((N_RULES, COLS_PER_WORKER), jnp.float32),
            pltpu.SemaphoreType.DMA,
            pltpu.SemaphoreType.DMA,
            pltpu.SemaphoreType.DMA,
            pltpu.SemaphoreType.DMA,
        ],
        compiler_params=pltpu.CompilerParams(needs_layout_passes=False,
                                             skip_device_barrier=True),
    )
    return k(xt)


def kernel(x):
    xt = x.transpose(1, 2, 0).reshape(N_IN, BATCH)
    return _run(xt).T


# speedup vs baseline: 1.0199x; 1.0030x over previous
import functools

import jax
import jax.numpy as jnp
from jax import lax
from jax.experimental import pallas as pl
from jax.experimental.pallas import tpu as pltpu
from jax.experimental.pallas import tpu_sc as plsc

BATCH = 16384
N_IN = 16
N_RULES = 64
NUM_CORES = 2
NUM_SUBCORES = 16
NUM_WORKERS = NUM_CORES * NUM_SUBCORES
COLS_PER_WORKER = BATCH // NUM_WORKERS
LANES = 16
GROUPS = COLS_PER_WORKER // LANES


HALF = COLS_PER_WORKER // 2
QUARTER = COLS_PER_WORKER // 4
Q_GROUPS = QUARTER // LANES


def _body(xt_hbm, outt_hbm, in_v, out_v, isem0, isem1, osem0, osem1):
    wid = lax.axis_index("s") * NUM_CORES + lax.axis_index("c")
    base = wid * COLS_PER_WORKER

    hin0 = pltpu.async_copy(xt_hbm.at[:, pl.ds(base, HALF)],
                            in_v.at[:, pl.ds(0, HALF)], isem0)
    hin1 = pltpu.async_copy(xt_hbm.at[:, pl.ds(base + HALF, HALF)],
                            in_v.at[:, pl.ds(HALF, HALF)], isem1)
    hin0.wait()

    def group(g, carry):
        col = g * LANES
        vals = [in_v[p, pl.ds(col, LANES)] for p in range(N_IN)]
        a, c = vals[:8], vals[8:]
        for i in range(8):
            for j in range(8):
                out_v[i * 8 + j, pl.ds(col, LANES)] = jnp.minimum(a[i], c[j])
        return carry

    lax.fori_loop(0, 2 * Q_GROUPS, group, 0)
    h0 = pltpu.async_copy(out_v.at[:, pl.ds(0, HALF)],
                          outt_hbm.at[:, pl.ds(base, HALF)], osem0)
    hin1.wait()
    lax.fori_loop(2 * Q_GROUPS, 4 * Q_GROUPS, group, 0)
    h1 = pltpu.async_copy(out_v.at[:, pl.ds(HALF, HALF)],
                          outt_hbm.at[:, pl.ds(base + HALF, HALF)], osem1)
    h0.wait()
    h1.wait()


@functools.partial(jax.jit, static_argnames=())
def _run(xt):
    mesh = plsc.VectorSubcoreMesh(
        core_axis_name="c", subcore_axis_name="s",
        num_cores=NUM_CORES, num_subcores=NUM_SUBCORES,
    )
    k = pl.kernel(
        _body,
        out_type=jax.ShapeDtypeStruct((N_RULES, BATCH), jnp.float32),
        mesh=mesh,
        scratch_types=[
            pltpu.VMEM((N_IN, COLS_PER_WORKER), jnp.float32),
            pltpu.VMEM((N_RULES, COLS_PER_WORKER), jnp.float32),
            pltpu.SemaphoreType.DMA,
            pltpu.SemaphoreType.DMA,
            pltpu.SemaphoreType.DMA,
            pltpu.SemaphoreType.DMA,
        ],
        compiler_params=pltpu.CompilerParams(needs_layout_passes=False),
    )
    return k(xt)


def kernel(x):
    xt = x.transpose(1, 2, 0).reshape(N_IN, BATCH)
    return _run(xt).T
